# baseline (speedup 1.0000x reference)
import functools

import jax
import jax.numpy as jnp
from jax import lax
from jax.experimental import pallas as pl
from jax.experimental.pallas import tpu as pltpu
from jax.experimental.pallas import tpu_sc as plsc

N = 10000
E = 320000
DF = 128
DE = 16
DT = 16
DA = DE + DT
DO = 128

NC = 2
NS = 16
NPAD = 10240
ROWS_PER_TILE = NPAD // NS
EDGES_PER_CORE = E // NC
EDGES_PER_TILE = EDGES_PER_CORE // NS
CHUNK = 80
NCHUNKS = EDGES_PER_TILE // CHUNK
NSLOTS = 2


def _sc_aggregate(row, col, x, eat):
  mesh = plsc.VectorSubcoreMesh(core_axis_name="c", subcore_axis_name="s")

  slot_scratch = []
  for _ in range(NSLOTS):
    slot_scratch += [
        pltpu.VMEM((CHUNK,), jnp.int32),
        pltpu.VMEM((CHUNK,), jnp.int32),
        pltpu.VMEM((CHUNK, DF), jnp.float32),
        pltpu.VMEM((CHUNK, DA), jnp.float32),
        pltpu.SemaphoreType.DMA,
        pltpu.SemaphoreType.DMA,
        pltpu.SemaphoreType.DMA,
    ]

  @functools.partial(
      pl.kernel,
      out_type=(
          jax.ShapeDtypeStruct((NC, NPAD, DF), jnp.float32),
          jax.ShapeDtypeStruct((NC, NPAD, DA), jnp.float32),
      ),
      mesh=mesh,
      compiler_params=pltpu.CompilerParams(use_tc_tiling_on_sc=False),
      scratch_types=slot_scratch + [
          pltpu.VMEM_SHARED((NPAD, DF), jnp.float32),
          pltpu.VMEM_SHARED((NPAD, DA), jnp.float32),
      ],
  )
  def agg_kernel(row_h, col_h, x_h, eat_h, px_h, pat_h, *sc):
    rows = [sc[7 * k + 0] for k in range(NSLOTS)]
    cols = [sc[7 * k + 1] for k in range(NSLOTS)]
    xrs = [sc[7 * k + 2] for k in range(NSLOTS)]
    eats = [sc[7 * k + 3] for k in range(NSLOTS)]
    sis = [sc[7 * k + 4] for k in range(NSLOTS)]
    sgs = [sc[7 * k + 5] for k in range(NSLOTS)]
    sss = [sc[7 * k + 6] for k in range(NSLOTS)]
    aggx = sc[7 * NSLOTS]
    aggat = sc[7 * NSLOTS + 1]

    c = lax.axis_index("c")
    s = lax.axis_index("s")
    rbase = s * ROWS_PER_TILE

    zeros16 = jnp.zeros((16,), jnp.float32)

    @pl.loop(0, CHUNK)
    def _(i):
      @pl.loop(0, DA // 16)
      def _(k):
        eats[0][i, pl.ds(k * 16, 16)] = zeros16

      @pl.loop(0, DF // 16)
      def _(k):
        xrs[0][i, pl.ds(k * 16, 16)] = zeros16

    @pl.loop(0, ROWS_PER_TILE // CHUNK)
    def _(k):
      pltpu.sync_copy(xrs[0], aggx.at[pl.ds(rbase + k * CHUNK, CHUNK)])
      pltpu.sync_copy(eats[0], aggat.at[pl.ds(rbase + k * CHUNK, CHUNK)])

    plsc.subcore_barrier()

    ebase = c * EDGES_PER_CORE + s * EDGES_PER_TILE
    last = NCHUNKS - 1

    def idx_start(j, b):
      off = ebase + jnp.minimum(j, last) * CHUNK
      pltpu.async_copy(row_h.at[pl.ds(off, CHUNK)], rows[b], sis[b])
      pltpu.async_copy(col_h.at[pl.ds(off, CHUNK)], cols[b], sis[b])
      pltpu.async_copy(eat_h.at[pl.ds(off, CHUNK)], eats[b], sis[b])

    def idx_wait(b):
      pltpu.make_async_copy(row_h.at[pl.ds(0, CHUNK)], rows[b], sis[b]).wait()
      pltpu.make_async_copy(col_h.at[pl.ds(0, CHUNK)], cols[b], sis[b]).wait()
      pltpu.make_async_copy(eat_h.at[pl.ds(0, CHUNK)], eats[b], sis[b]).wait()

    def gather_start(b):
      pltpu.async_copy(x_h.at[cols[b]], xrs[b], sgs[b])

    def gather_wait(b):
      pltpu.make_async_copy(x_h.at[cols[b]], xrs[b], sgs[b]).wait()

    def scat_start(b):
      pltpu.async_copy(xrs[b], aggx.at[rows[b]], sss[b], add=True)
      pltpu.async_copy(eats[b], aggat.at[rows[b]], sss[b], add=True)

    def scat_wait(b):
      pltpu.make_async_copy(xrs[b], aggx.at[rows[b]], sss[b]).wait()
      pltpu.make_async_copy(eats[b], aggat.at[rows[b]], sss[b]).wait()

    idx_start(0, 0)
    idx_wait(0)
    gather_start(0)
    gather_wait(0)
    idx_start(1, 1)
    scat_start(0)

    @pl.loop(0, (NCHUNKS - 1) // 2)
    def _(i):
      j = 2 * i
      idx_wait(1)
      gather_start(1)
      gather_wait(1)
      scat_wait(0)
      idx_start(j + 2, 0)
      scat_start(1)
      idx_wait(0)
      gather_start(0)
      gather_wait(0)
      scat_wait(1)
      idx_start(jnp.minimum(j + 3, NCHUNKS - 1), 1)
      scat_start(0)

    scat_wait(0)
    idx_wait(1)

    plsc.subcore_barrier()
    pltpu.sync_copy(aggx.at[pl.ds(rbase, ROWS_PER_TILE)],
                    px_h.at[c, pl.ds(rbase, ROWS_PER_TILE)])
    pltpu.sync_copy(aggat.at[pl.ds(rbase, ROWS_PER_TILE)],
                    pat_h.at[c, pl.ds(rbase, ROWS_PER_TILE)])

  return agg_kernel(row, col, x, eat)


BLK = 1000


def _tc_concat(edge_attr, edge_t):
  eblk = 8000

  def body(ea_r, et_r, o_r):
    o_r[:, :DE] = ea_r[...]
    o_r[:, DE:] = et_r[...]

  return pl.pallas_call(
      body,
      grid=(E // eblk,),
      in_specs=[
          pl.BlockSpec((eblk, DE), lambda i: (i, 0)),
          pl.BlockSpec((eblk, DT), lambda i: (i, 0)),
      ],
      out_specs=pl.BlockSpec((eblk, DA), lambda i: (i, 0)),
      out_shape=jax.ShapeDtypeStruct((E, DA), jnp.float32),
  )(edge_attr, edge_t)


def _tc_combine(px, pat, x, wmx, wmat, wr, bm, br):
  def body(px_r, pat_r, x_r, wmx_r, wmat_r, wr_r, bm_r, br_r, o_r):
    aggx = px_r[0] + px_r[1]
    aggat = pat_r[0] + pat_r[1]
    acc = jnp.dot(aggx, wmx_r[...], preferred_element_type=jnp.float32)
    acc = acc + jnp.dot(aggat, wmat_r[...], preferred_element_type=jnp.float32)
    acc = 0.5 * (acc + bm_r[...])
    acc = acc + jnp.dot(x_r[...], wr_r[...], preferred_element_type=jnp.float32)
    o_r[...] = acc + br_r[...]

  return pl.pallas_call(
      body,
      grid=(N // BLK,),
      in_specs=[
          pl.BlockSpec((NC, BLK, DF), lambda i: (0, i, 0)),
          pl.BlockSpec((NC, BLK, DA), lambda i: (0, i, 0)),
          pl.BlockSpec((BLK, DF), lambda i: (i, 0)),
          pl.BlockSpec((DF, DO), lambda i: (0, 0)),
          pl.BlockSpec((DA, DO), lambda i: (0, 0)),
          pl.BlockSpec((DF, DO), lambda i: (0, 0)),
          pl.BlockSpec((1, DO), lambda i: (0, 0)),
          pl.BlockSpec((1, DO), lambda i: (0, 0)),
      ],
      out_specs=pl.BlockSpec((BLK, DO), lambda i: (i, 0)),
      out_shape=jax.ShapeDtypeStruct((N, DO), jnp.float32),
  )(px, pat, x, wmx, wmat, wr, bm, br)


def kernel(x, edge_index, edge_attr, edge_t, W_m, b_m, W_r, b_r):
  ei = edge_index.astype(jnp.int32)
  eat = jnp.concatenate([edge_attr, edge_t], axis=1)
  px, pat = _sc_aggregate(ei[0], ei[1], x, eat)
  wmx = W_m[:DF]
  wmat = W_m[DF:]
  bm = b_m.reshape(1, DO)
  br = b_r.reshape(1, DO)
  return _tc_combine(px, pat, x, wmx, wmat, W_r, bm, br)


# speedup vs baseline: 1.0815x; 1.0815x over previous
import functools

import jax
import jax.numpy as jnp
from jax import lax
from jax.experimental import pallas as pl
from jax.experimental.pallas import tpu as pltpu
from jax.experimental.pallas import tpu_sc as plsc

N = 10000
E = 320000
DF = 128
DE = 16
DT = 16
DA = DE + DT
DO = 128

NC = 2
NS = 16
NPAD = 10240
ROWS_PER_TILE = NPAD // NS
EDGES_PER_CORE = E // NC
EDGES_PER_TILE = EDGES_PER_CORE // NS
CHUNK = 80
NCHUNKS = EDGES_PER_TILE // CHUNK
NSLOTS = 2


def _sc_aggregate(row, col, x, eat):
  mesh = plsc.VectorSubcoreMesh(core_axis_name="c", subcore_axis_name="s")

  slot_scratch = []
  for _ in range(NSLOTS):
    slot_scratch += [
        pltpu.VMEM((CHUNK,), jnp.int32),
        pltpu.VMEM((CHUNK,), jnp.int32),
        pltpu.VMEM((CHUNK, DF), jnp.float32),
        pltpu.VMEM((CHUNK, DA), jnp.float32),
        pltpu.SemaphoreType.DMA,
        pltpu.SemaphoreType.DMA,
        pltpu.SemaphoreType.DMA,
    ]

  @functools.partial(
      pl.kernel,
      out_type=(
          jax.ShapeDtypeStruct((NC, NPAD, DF), jnp.float32),
          jax.ShapeDtypeStruct((NC, NPAD, DA), jnp.float32),
      ),
      mesh=mesh,
      compiler_params=pltpu.CompilerParams(use_tc_tiling_on_sc=False),
      scratch_types=slot_scratch + [
          pltpu.VMEM_SHARED((NPAD, DF), jnp.float32),
          pltpu.VMEM_SHARED((NPAD, DA), jnp.float32),
      ],
  )
  def agg_kernel(row_h, col_h, x_h, eat_h, px_h, pat_h, *sc):
    rows = [sc[7 * k + 0] for k in range(NSLOTS)]
    cols = [sc[7 * k + 1] for k in range(NSLOTS)]
    xrs = [sc[7 * k + 2] for k in range(NSLOTS)]
    eats = [sc[7 * k + 3] for k in range(NSLOTS)]
    sis = [sc[7 * k + 4] for k in range(NSLOTS)]
    sgs = [sc[7 * k + 5] for k in range(NSLOTS)]
    sss = [sc[7 * k + 6] for k in range(NSLOTS)]
    aggx = sc[7 * NSLOTS]
    aggat = sc[7 * NSLOTS + 1]

    c = lax.axis_index("c")
    s = lax.axis_index("s")
    rbase = s * ROWS_PER_TILE

    zeros16 = jnp.zeros((16,), jnp.float32)

    @pl.loop(0, CHUNK)
    def _(i):
      @pl.loop(0, DA // 16)
      def _(k):
        eats[0][i, pl.ds(k * 16, 16)] = zeros16

      @pl.loop(0, DF // 16)
      def _(k):
        xrs[0][i, pl.ds(k * 16, 16)] = zeros16

    @pl.loop(0, ROWS_PER_TILE // CHUNK)
    def _(k):
      pltpu.sync_copy(xrs[0], aggx.at[pl.ds(rbase + k * CHUNK, CHUNK)])
      pltpu.sync_copy(eats[0], aggat.at[pl.ds(rbase + k * CHUNK, CHUNK)])

    plsc.subcore_barrier()

    ebase = c * EDGES_PER_CORE + s * EDGES_PER_TILE
    last = NCHUNKS - 1

    def idx_start(j, b):
      off = ebase + jnp.minimum(j, last) * CHUNK
      pltpu.async_copy(row_h.at[pl.ds(off, CHUNK)], rows[b], sis[b])
      pltpu.async_copy(col_h.at[pl.ds(off, CHUNK)], cols[b], sis[b])
      pltpu.async_copy(eat_h.at[pl.ds(off, CHUNK)], eats[b], sis[b])

    def idx_wait(b):
      pltpu.make_async_copy(row_h.at[pl.ds(0, CHUNK)], rows[b], sis[b]).wait()
      pltpu.make_async_copy(col_h.at[pl.ds(0, CHUNK)], cols[b], sis[b]).wait()
      pltpu.make_async_copy(eat_h.at[pl.ds(0, CHUNK)], eats[b], sis[b]).wait()

    def gather_start(b):
      pltpu.async_copy(x_h.at[cols[b]], xrs[b], sgs[b])

    def gather_wait(b):
      pltpu.make_async_copy(x_h.at[cols[b]], xrs[b], sgs[b]).wait()

    def scat_start(b):
      pltpu.async_copy(xrs[b], aggx.at[rows[b]], sss[b], add=True)
      pltpu.async_copy(eats[b], aggat.at[rows[b]], sss[b], add=True)

    def scat_wait(b):
      pltpu.make_async_copy(xrs[b], aggx.at[rows[b]], sss[b]).wait()
      pltpu.make_async_copy(eats[b], aggat.at[rows[b]], sss[b]).wait()

    idx_start(0, 0)
    idx_start(1, 1)

    @pl.loop(0, (NCHUNKS - 1) // 2)
    def _(i):
      a = 2 * i
      idx_wait(0)
      gather_start(0)
      idx_wait(1)
      gather_start(1)
      gather_wait(0)
      scat_start(0)
      gather_wait(1)
      scat_start(1)
      scat_wait(0)
      idx_start(a + 2, 0)
      scat_wait(1)
      idx_start(jnp.minimum(a + 3, NCHUNKS - 1), 1)

    idx_wait(0)
    gather_start(0)
    gather_wait(0)
    scat_start(0)
    scat_wait(0)
    idx_wait(1)

    plsc.subcore_barrier()
    pltpu.sync_copy(aggx.at[pl.ds(rbase, ROWS_PER_TILE)],
                    px_h.at[c, pl.ds(rbase, ROWS_PER_TILE)])
    pltpu.sync_copy(aggat.at[pl.ds(rbase, ROWS_PER_TILE)],
                    pat_h.at[c, pl.ds(rbase, ROWS_PER_TILE)])

  return agg_kernel(row, col, x, eat)


BLK = 1000


def _tc_concat(edge_attr, edge_t):
  eblk = 8000

  def body(ea_r, et_r, o_r):
    o_r[:, :DE] = ea_r[...]
    o_r[:, DE:] = et_r[...]

  return pl.pallas_call(
      body,
      grid=(E // eblk,),
      in_specs=[
          pl.BlockSpec((eblk, DE), lambda i: (i, 0)),
          pl.BlockSpec((eblk, DT), lambda i: (i, 0)),
      ],
      out_specs=pl.BlockSpec((eblk, DA), lambda i: (i, 0)),
      out_shape=jax.ShapeDtypeStruct((E, DA), jnp.float32),
  )(edge_attr, edge_t)


def _tc_combine(px, pat, x, wmx, wmat, wr, bm, br):
  def body(px_r, pat_r, x_r, wmx_r, wmat_r, wr_r, bm_r, br_r, o_r):
    aggx = px_r[0] + px_r[1]
    aggat = pat_r[0] + pat_r[1]
    acc = jnp.dot(aggx, wmx_r[...], preferred_element_type=jnp.float32)
    acc = acc + jnp.dot(aggat, wmat_r[...], preferred_element_type=jnp.float32)
    acc = 0.5 * (acc + bm_r[...])
    acc = acc + jnp.dot(x_r[...], wr_r[...], preferred_element_type=jnp.float32)
    o_r[...] = acc + br_r[...]

  return pl.pallas_call(
      body,
      grid=(N // BLK,),
      in_specs=[
          pl.BlockSpec((NC, BLK, DF), lambda i: (0, i, 0)),
          pl.BlockSpec((NC, BLK, DA), lambda i: (0, i, 0)),
          pl.BlockSpec((BLK, DF), lambda i: (i, 0)),
          pl.BlockSpec((DF, DO), lambda i: (0, 0)),
          pl.BlockSpec((DA, DO), lambda i: (0, 0)),
          pl.BlockSpec((DF, DO), lambda i: (0, 0)),
          pl.BlockSpec((1, DO), lambda i: (0, 0)),
          pl.BlockSpec((1, DO), lambda i: (0, 0)),
      ],
      out_specs=pl.BlockSpec((BLK, DO), lambda i: (i, 0)),
      out_shape=jax.ShapeDtypeStruct((N, DO), jnp.float32),
  )(px, pat, x, wmx, wmat, wr, bm, br)


def kernel(x, edge_index, edge_attr, edge_t, W_m, b_m, W_r, b_r):
  ei = edge_index.astype(jnp.int32)
  eat = jnp.concatenate([edge_attr, edge_t], axis=1)
  px, pat = _sc_aggregate(ei[0], ei[1], x, eat)
  wmx = W_m[:DF]
  wmat = W_m[DF:]
  bm = b_m.reshape(1, DO)
  br = b_r.reshape(1, DO)
  return _tc_combine(px, pat, x, wmx, wmat, W_r, bm, br)


# speedup vs baseline: 1.0935x; 1.0111x over previous
import functools

import jax
import jax.numpy as jnp
from jax import lax
from jax.experimental import pallas as pl
from jax.experimental.pallas import tpu as pltpu
from jax.experimental.pallas import tpu_sc as plsc

N = 10000
E = 320000
DF = 128
DE = 16
DT = 16
DA = DE + DT
DO = 128

NC = 2
NS = 16
NPAD = 10240
ROWS_PER_TILE = NPAD // NS
EDGES_PER_CORE = E // NC
EDGES_PER_TILE = EDGES_PER_CORE // NS
CHUNK = 88
NCHUNKS = EDGES_PER_TILE // CHUNK
TAIL = EDGES_PER_TILE - NCHUNKS * CHUNK
NSLOTS = 2


def _sc_aggregate(row, col, x, eat):
  mesh = plsc.VectorSubcoreMesh(core_axis_name="c", subcore_axis_name="s")

  slot_scratch = []
  for _ in range(NSLOTS):
    slot_scratch += [
        pltpu.VMEM((CHUNK,), jnp.int32),
        pltpu.VMEM((CHUNK,), jnp.int32),
        pltpu.VMEM((CHUNK, DF), jnp.float32),
        pltpu.VMEM((CHUNK, DA), jnp.float32),
        pltpu.SemaphoreType.DMA,
        pltpu.SemaphoreType.DMA,
        pltpu.SemaphoreType.DMA,
    ]

  @functools.partial(
      pl.kernel,
      out_type=(
          jax.ShapeDtypeStruct((NC, NPAD, DF), jnp.float32),
          jax.ShapeDtypeStruct((NC, NPAD, DA), jnp.float32),
      ),
      mesh=mesh,
      compiler_params=pltpu.CompilerParams(use_tc_tiling_on_sc=False),
      scratch_types=slot_scratch + [
          pltpu.VMEM_SHARED((NPAD, DF), jnp.float32),
          pltpu.VMEM_SHARED((NPAD, DA), jnp.float32),
      ],
  )
  def agg_kernel(row_h, col_h, x_h, eat_h, px_h, pat_h, *sc):
    rows = [sc[7 * k + 0] for k in range(NSLOTS)]
    cols = [sc[7 * k + 1] for k in range(NSLOTS)]
    xrs = [sc[7 * k + 2] for k in range(NSLOTS)]
    eats = [sc[7 * k + 3] for k in range(NSLOTS)]
    sis = [sc[7 * k + 4] for k in range(NSLOTS)]
    sgs = [sc[7 * k + 5] for k in range(NSLOTS)]
    sss = [sc[7 * k + 6] for k in range(NSLOTS)]
    aggx = sc[7 * NSLOTS]
    aggat = sc[7 * NSLOTS + 1]

    c = lax.axis_index("c")
    s = lax.axis_index("s")
    rbase = s * ROWS_PER_TILE

    zeros16 = jnp.zeros((16,), jnp.float32)

    @pl.loop(0, 80)
    def _(i):
      @pl.loop(0, DA // 16)
      def _(k):
        eats[0][i, pl.ds(k * 16, 16)] = zeros16

      @pl.loop(0, DF // 16)
      def _(k):
        xrs[0][i, pl.ds(k * 16, 16)] = zeros16

    @pl.loop(0, ROWS_PER_TILE // 80)
    def _(k):
      pltpu.sync_copy(xrs[0].at[pl.ds(0, 80)],
                      aggx.at[pl.ds(rbase + k * 80, 80)])
      pltpu.sync_copy(eats[0].at[pl.ds(0, 80)],
                      aggat.at[pl.ds(rbase + k * 80, 80)])

    plsc.subcore_barrier()

    ebase = c * EDGES_PER_CORE + s * EDGES_PER_TILE
    last = NCHUNKS - 1

    def idx_start(j, b, n=CHUNK):
      off = ebase + jnp.minimum(j, last) * CHUNK
      pltpu.async_copy(row_h.at[pl.ds(off, n)], rows[b].at[pl.ds(0, n)], sis[b])
      pltpu.async_copy(col_h.at[pl.ds(off, n)], cols[b].at[pl.ds(0, n)], sis[b])
      pltpu.async_copy(eat_h.at[pl.ds(off, n)], eats[b].at[pl.ds(0, n)], sis[b])

    def idx_wait(b, n=CHUNK):
      pltpu.make_async_copy(row_h.at[pl.ds(0, n)], rows[b].at[pl.ds(0, n)],
                            sis[b]).wait()
      pltpu.make_async_copy(col_h.at[pl.ds(0, n)], cols[b].at[pl.ds(0, n)],
                            sis[b]).wait()
      pltpu.make_async_copy(eat_h.at[pl.ds(0, n)], eats[b].at[pl.ds(0, n)],
                            sis[b]).wait()

    def tail_start(b):
      off = ebase + NCHUNKS * CHUNK
      pltpu.async_copy(row_h.at[pl.ds(off, TAIL)], rows[b].at[pl.ds(0, TAIL)],
                       sis[b])
      pltpu.async_copy(col_h.at[pl.ds(off, TAIL)], cols[b].at[pl.ds(0, TAIL)],
                       sis[b])
      pltpu.async_copy(eat_h.at[pl.ds(off, TAIL)], eats[b].at[pl.ds(0, TAIL)],
                       sis[b])

    def gather_start(b, n=CHUNK):
      pltpu.async_copy(x_h.at[cols[b].at[pl.ds(0, n)]],
                       xrs[b].at[pl.ds(0, n)], sgs[b])

    def gather_wait(b, n=CHUNK):
      pltpu.make_async_copy(x_h.at[cols[b].at[pl.ds(0, n)]],
                            xrs[b].at[pl.ds(0, n)], sgs[b]).wait()

    def scat_start(b, n=CHUNK):
      pltpu.async_copy(xrs[b].at[pl.ds(0, n)],
                       aggx.at[rows[b].at[pl.ds(0, n)]], sss[b], add=True)
      pltpu.async_copy(eats[b].at[pl.ds(0, n)],
                       aggat.at[rows[b].at[pl.ds(0, n)]], sss[b], add=True)

    def scat_wait(b, n=CHUNK):
      pltpu.make_async_copy(xrs[b].at[pl.ds(0, n)],
                            aggx.at[rows[b].at[pl.ds(0, n)]], sss[b]).wait()
      pltpu.make_async_copy(eats[b].at[pl.ds(0, n)],
                            aggat.at[rows[b].at[pl.ds(0, n)]], sss[b]).wait()

    idx_start(0, 0)
    idx_start(1, 1)

    @pl.loop(0, (NCHUNKS - 1) // 2)
    def _(i):
      a = 2 * i
      idx_wait(0)
      gather_start(0)
      idx_wait(1)
      gather_start(1)
      gather_wait(0)
      scat_start(0)
      gather_wait(1)
      scat_start(1)
      scat_wait(0)
      idx_start(a + 2, 0)
      scat_wait(1)
      idx_start(jnp.minimum(a + 3, NCHUNKS - 1), 1)

    idx_wait(0)
    gather_start(0)
    idx_wait(1)
    tail_start(1)
    gather_wait(0)
    scat_start(0)
    idx_wait(1, TAIL)
    gather_start(1, TAIL)
    gather_wait(1, TAIL)
    scat_wait(0)
    scat_start(1, TAIL)
    scat_wait(1, TAIL)

    plsc.subcore_barrier()
    pltpu.sync_copy(aggx.at[pl.ds(rbase, ROWS_PER_TILE)],
                    px_h.at[c, pl.ds(rbase, ROWS_PER_TILE)])
    pltpu.sync_copy(aggat.at[pl.ds(rbase, ROWS_PER_TILE)],
                    pat_h.at[c, pl.ds(rbase, ROWS_PER_TILE)])

  return agg_kernel(row, col, x, eat)


BLK = 1000


def _tc_concat(edge_attr, edge_t):
  eblk = 8000

  def body(ea_r, et_r, o_r):
    o_r[:, :DE] = ea_r[...]
    o_r[:, DE:] = et_r[...]

  return pl.pallas_call(
      body,
      grid=(E // eblk,),
      in_specs=[
          pl.BlockSpec((eblk, DE), lambda i: (i, 0)),
          pl.BlockSpec((eblk, DT), lambda i: (i, 0)),
      ],
      out_specs=pl.BlockSpec((eblk, DA), lambda i: (i, 0)),
      out_shape=jax.ShapeDtypeStruct((E, DA), jnp.float32),
  )(edge_attr, edge_t)


def _tc_combine(px, pat, x, wmx, wmat, wr, bm, br):
  def body(px_r, pat_r, x_r, wmx_r, wmat_r, wr_r, bm_r, br_r, o_r):
    aggx = px_r[0] + px_r[1]
    aggat = pat_r[0] + pat_r[1]
    acc = jnp.dot(aggx, wmx_r[...], preferred_element_type=jnp.float32)
    acc = acc + jnp.dot(aggat, wmat_r[...], preferred_element_type=jnp.float32)
    acc = 0.5 * (acc + bm_r[...])
    acc = acc + jnp.dot(x_r[...], wr_r[...], preferred_element_type=jnp.float32)
    o_r[...] = acc + br_r[...]

  return pl.pallas_call(
      body,
      grid=(N // BLK,),
      in_specs=[
          pl.BlockSpec((NC, BLK, DF), lambda i: (0, i, 0)),
          pl.BlockSpec((NC, BLK, DA), lambda i: (0, i, 0)),
          pl.BlockSpec((BLK, DF), lambda i: (i, 0)),
          pl.BlockSpec((DF, DO), lambda i: (0, 0)),
          pl.BlockSpec((DA, DO), lambda i: (0, 0)),
          pl.BlockSpec((DF, DO), lambda i: (0, 0)),
          pl.BlockSpec((1, DO), lambda i: (0, 0)),
          pl.BlockSpec((1, DO), lambda i: (0, 0)),
      ],
      out_specs=pl.BlockSpec((BLK, DO), lambda i: (i, 0)),
      out_shape=jax.ShapeDtypeStruct((N, DO), jnp.float32),
  )(px, pat, x, wmx, wmat, wr, bm, br)


def kernel(x, edge_index, edge_attr, edge_t, W_m, b_m, W_r, b_r):
  ei = edge_index.astype(jnp.int32)
  eat = jnp.concatenate([edge_attr, edge_t], axis=1)
  px, pat = _sc_aggregate(ei[0], ei[1], x, eat)
  wmx = W_m[:DF]
  wmat = W_m[DF:]
  bm = b_m.reshape(1, DO)
  br = b_r.reshape(1, DO)
  return _tc_combine(px, pat, x, wmx, wmat, W_r, bm, br)
